# Initial kernel scaffold; baseline (speedup 1.0000x reference)
import jax
import jax.numpy as jnp
from jax.experimental import pallas as pl


def kernel(x, edge_index, W1, b1, W2, b2):
    raise NotImplementedError("write your pallas kernel here")


# speedup vs baseline: 20.0351x; 20.0351x over previous
import functools

import jax
import jax.numpy as jnp
from jax import lax
from jax.experimental import pallas as pl
from jax.experimental.pallas import tpu as pltpu
from jax.experimental.pallas import tpu_sc as plsc

N_NODES = 10000
D = 128
N_EDGES = 320000

NC = 2
NS = 16
NW = NC * NS
EPW = N_EDGES // NW
CH = 80
NCHUNK = EPW // CH
NPS = N_NODES // NS
ZR = 125
RB = 1000
G = N_NODES // RB

@functools.cache
def _make_degree_hist():
    mesh = plsc.VectorSubcoreMesh(core_axis_name="c", subcore_axis_name="s")
    return pl.kernel(
        _degree_hist_body,
        mesh=mesh,
        compiler_params=pltpu.CompilerParams(
            needs_layout_passes=False, use_tc_tiling_on_sc=False),
        out_type=jax.ShapeDtypeStruct((NW, N_NODES), jnp.float32),
        scratch_types=[
            pltpu.VMEM((EPW,), jnp.int32),
            pltpu.VMEM((N_NODES,), jnp.float32),
        ],
    )


def _degree_hist_body(dst_hbm, out_hbm, dst_v, hist_v):
    cid = lax.axis_index("c")
    sid = lax.axis_index("s")
    wid = sid * NC + cid
    pltpu.sync_copy(dst_hbm.at[pl.ds(wid * EPW, EPW)], dst_v)

    zeros = jnp.zeros((16,), jnp.float32)

    def zero_body(i, _):
        hist_v[pl.ds(i * 16, 16)] = zeros
        return ()

    lax.fori_loop(0, N_NODES // 16, zero_body, ())

    ones = jnp.ones((16,), jnp.float32)

    def body(i, _):
        idx = dst_v[pl.ds(i * 16, 16)]
        plsc.addupdate_scatter(hist_v, [idx], ones)
        return ()

    lax.fori_loop(0, EPW // 16, body, ())
    pltpu.sync_copy(hist_v, out_hbm.at[wid])


@functools.cache
def _make_edge_aggregate():
    mesh = plsc.VectorSubcoreMesh(core_axis_name="c", subcore_axis_name="s")
    return pl.kernel(
        _edge_aggregate_body,
        mesh=mesh,
        compiler_params=pltpu.CompilerParams(
            needs_layout_passes=False, use_tc_tiling_on_sc=False),
        out_type=jax.ShapeDtypeStruct((NC, N_NODES, D), jnp.float32),
        scratch_types=[
            pltpu.VMEM((NCHUNK, CH), jnp.int32),
            pltpu.VMEM((NCHUNK, CH), jnp.int32),
            pltpu.VMEM((CH, D), jnp.float32),
            pltpu.VMEM((ZR, D), jnp.float32),
            pltpu.VMEM_SHARED((N_NODES, D), jnp.float32),
            pltpu.SemaphoreType.DMA,
        ],
    )


def _edge_aggregate_body(h_hbm, src_hbm, dst_hbm, out_hbm,
                         src_v, dst_v, rows_v, zero_v, acc_sh, sem):
    cid = lax.axis_index("c")
    sid = lax.axis_index("s")
    wid = sid * NC + cid

    pltpu.sync_copy(src_hbm.at[pl.ds(wid * NCHUNK, NCHUNK)], src_v)
    pltpu.sync_copy(dst_hbm.at[pl.ds(wid * NCHUNK, NCHUNK)], dst_v)

    zeros = jnp.zeros((16,), jnp.float32)

    def zero_body(i, _):
        zero_v[i // 8, pl.ds((i % 8) * 16, 16)] = zeros
        return ()

    lax.fori_loop(0, ZR * (D // 16), zero_body, ())
    for k in range(NPS // ZR):
        pltpu.sync_copy(zero_v, acc_sh.at[pl.ds(sid * NPS + k * ZR, ZR)])
    plsc.subcore_barrier()

    def chunk(j, _):
        pltpu.async_copy(h_hbm.at[src_v.at[j]], rows_v, sem).wait()
        pltpu.sync_copy(rows_v, acc_sh.at[dst_v.at[j]], add=True)
        return ()

    lax.fori_loop(0, NCHUNK, chunk, ())
    plsc.subcore_barrier()
    pltpu.sync_copy(acc_sh.at[pl.ds(sid * NPS, NPS)],
                    out_hbm.at[cid, pl.ds(sid * NPS, NPS)])


def _tc1_body(hist_ref, x_ref, w_ref, dinv_ref, hs_ref):
    deg = jnp.sum(hist_ref[...], axis=1, keepdims=True) + 1.0
    dinv = lax.rsqrt(deg)
    h = jnp.dot(x_ref[...], w_ref[...], preferred_element_type=jnp.float32)
    dinv_ref[...] = dinv
    hs_ref[...] = h * dinv


_tc1 = pl.pallas_call(
    _tc1_body,
    grid=(G,),
    in_specs=[
        pl.BlockSpec((RB, NW), lambda i: (i, 0)),
        pl.BlockSpec((RB, D), lambda i: (i, 0)),
        pl.BlockSpec((D, D), lambda i: (0, 0)),
    ],
    out_specs=[
        pl.BlockSpec((RB, 1), lambda i: (i, 0)),
        pl.BlockSpec((RB, D), lambda i: (i, 0)),
    ],
    out_shape=[
        jax.ShapeDtypeStruct((N_NODES, 1), jnp.float32),
        jax.ShapeDtypeStruct((N_NODES, D), jnp.float32),
    ],
)


def _tc2_body(agg_ref, hs_ref, dinv_ref, b_ref, w_ref, out_ref):
    dinv = dinv_ref[...]
    s = agg_ref[0] + agg_ref[1] + hs_ref[...]
    z = jnp.maximum(dinv * s + b_ref[...], 0.0)
    h2 = jnp.dot(z, w_ref[...], preferred_element_type=jnp.float32)
    out_ref[...] = h2 * dinv


_tc2 = pl.pallas_call(
    _tc2_body,
    grid=(G,),
    in_specs=[
        pl.BlockSpec((NC, RB, D), lambda i: (0, i, 0)),
        pl.BlockSpec((RB, D), lambda i: (i, 0)),
        pl.BlockSpec((RB, 1), lambda i: (i, 0)),
        pl.BlockSpec((1, D), lambda i: (0, 0)),
        pl.BlockSpec((D, D), lambda i: (0, 0)),
    ],
    out_specs=pl.BlockSpec((RB, D), lambda i: (i, 0)),
    out_shape=jax.ShapeDtypeStruct((N_NODES, D), jnp.float32),
)


def _tc3_body(agg_ref, hs_ref, dinv_ref, b_ref, out_ref):
    dinv = dinv_ref[...]
    out_ref[...] = dinv * (agg_ref[0] + agg_ref[1] + hs_ref[...]) + b_ref[...]


_tc3 = pl.pallas_call(
    _tc3_body,
    grid=(G,),
    in_specs=[
        pl.BlockSpec((NC, RB, D), lambda i: (0, i, 0)),
        pl.BlockSpec((RB, D), lambda i: (i, 0)),
        pl.BlockSpec((RB, 1), lambda i: (i, 0)),
        pl.BlockSpec((1, D), lambda i: (0, 0)),
    ],
    out_specs=pl.BlockSpec((RB, D), lambda i: (i, 0)),
    out_shape=jax.ShapeDtypeStruct((N_NODES, D), jnp.float32),
)


def kernel(x, edge_index, W1, b1, W2, b2):
    src = edge_index[0].astype(jnp.int32).reshape(NW * NCHUNK, CH)
    dst = edge_index[1].astype(jnp.int32).reshape(NW * NCHUNK, CH)
    dst_flat = edge_index[1].astype(jnp.int32)

    degree_hist = _make_degree_hist()
    edge_aggregate = _make_edge_aggregate()

    hist = degree_hist(dst_flat)
    dinv, hs1 = _tc1(hist.T, x, W1)
    agg1 = edge_aggregate(hs1, src, dst)
    hs2 = _tc2(agg1, hs1, dinv, b1.reshape(1, D), W2)
    agg2 = edge_aggregate(hs2, src, dst)
    out = _tc3(agg2, hs2, dinv, b2.reshape(1, D))
    return out


# speedup vs baseline: 31.2636x; 1.5604x over previous
import functools

import jax
import jax.numpy as jnp
from jax import lax
from jax.experimental import pallas as pl
from jax.experimental.pallas import tpu as pltpu
from jax.experimental.pallas import tpu_sc as plsc

N_NODES = 10000
D = 128
N_EDGES = 320000

NC = 2
NS = 16
NW = NC * NS
EPW = N_EDGES // NW
CH = 80
NCHUNK = EPW // CH
NPS = N_NODES // NS
RB = 1000
G = N_NODES // RB

@functools.cache
def _make_degree_hist():
    mesh = plsc.VectorSubcoreMesh(core_axis_name="c", subcore_axis_name="s")
    return pl.kernel(
        _degree_hist_body,
        mesh=mesh,
        compiler_params=pltpu.CompilerParams(
            needs_layout_passes=False, use_tc_tiling_on_sc=False),
        out_type=jax.ShapeDtypeStruct((NW, N_NODES), jnp.float32),
        scratch_types=[
            pltpu.VMEM((EPW,), jnp.int32),
            pltpu.VMEM((N_NODES,), jnp.float32),
        ],
    )


def _degree_hist_body(dst_hbm, out_hbm, dst_v, hist_v):
    cid = lax.axis_index("c")
    sid = lax.axis_index("s")
    wid = sid * NC + cid
    pltpu.sync_copy(dst_hbm.at[pl.ds(wid * EPW, EPW)], dst_v)

    zeros = jnp.zeros((16,), jnp.float32)

    def zero_body(i, _):
        hist_v[pl.ds(i * 16, 16)] = zeros
        return ()

    lax.fori_loop(0, N_NODES // 16, zero_body, ())

    ones = jnp.ones((16,), jnp.float32)

    def body(i, _):
        idx = dst_v[pl.ds(i * 16, 16)]
        plsc.addupdate_scatter(hist_v, [idx], ones)
        return ()

    lax.fori_loop(0, EPW // 16, body, ())
    pltpu.sync_copy(hist_v, out_hbm.at[wid])


@functools.cache
def _make_edge_aggregate():
    mesh = plsc.VectorSubcoreMesh(core_axis_name="c", subcore_axis_name="s")
    return pl.kernel(
        _edge_aggregate_body,
        mesh=mesh,
        compiler_params=pltpu.CompilerParams(
            needs_layout_passes=False, use_tc_tiling_on_sc=False),
        out_type=jax.ShapeDtypeStruct((NC, N_NODES, D), jnp.float32),
        scratch_types=[
            pltpu.VMEM((NCHUNK, CH), jnp.int32),
            pltpu.VMEM((NCHUNK, CH), jnp.int32),
            pltpu.VMEM((CH, D), jnp.float32),
            pltpu.VMEM((CH, D), jnp.float32),
            pltpu.VMEM_SHARED((N_NODES, D), jnp.float32),
            pltpu.SemaphoreType.DMA,
            pltpu.SemaphoreType.DMA,
        ],
    )


def _edge_aggregate_body(h_hbm, src_hbm, dst_hbm, out_hbm,
                         src_v, dst_v, rows_a, rows_b, acc_sh,
                         sem_a, sem_b):
    cid = lax.axis_index("c")
    sid = lax.axis_index("s")
    wid = sid * NC + cid

    pltpu.sync_copy(src_hbm.at[pl.ds(wid * NCHUNK, NCHUNK)], src_v)
    pltpu.sync_copy(dst_hbm.at[pl.ds(wid * NCHUNK, NCHUNK)], dst_v)

    zeros = jnp.zeros((16,), jnp.float32)

    def zero_body(i, _):
        rows_a[i // 8, pl.ds((i % 8) * 16, 16)] = zeros
        return ()

    lax.fori_loop(0, CH * (D // 16), zero_body, ())
    for k in range(NPS // CH):
        pltpu.sync_copy(rows_a, acc_sh.at[pl.ds(sid * NPS + k * CH, CH)])
    tail = NPS - (NPS // CH) * CH
    if tail:
        pltpu.sync_copy(rows_a.at[pl.ds(0, tail)],
                        acc_sh.at[pl.ds(sid * NPS + (NPS // CH) * CH, tail)])
    plsc.subcore_barrier()

    def wait_gather(buf, sem):
        pltpu.make_async_copy(h_hbm.at[src_v.at[0]], buf, sem).wait()

    pltpu.async_copy(h_hbm.at[src_v.at[0]], rows_a, sem_a)

    def chunk_pair(i, _):
        j0 = i * 2
        pltpu.async_copy(h_hbm.at[src_v.at[j0 + 1]], rows_b, sem_b)
        wait_gather(rows_a, sem_a)
        pltpu.sync_copy(rows_a, acc_sh.at[dst_v.at[j0]], add=True)
        pltpu.async_copy(h_hbm.at[src_v.at[j0 + 2]], rows_a, sem_a)
        wait_gather(rows_b, sem_b)
        pltpu.sync_copy(rows_b, acc_sh.at[dst_v.at[j0 + 1]], add=True)
        return ()

    lax.fori_loop(0, (NCHUNK - 1) // 2, chunk_pair, ())
    wait_gather(rows_a, sem_a)
    pltpu.sync_copy(rows_a, acc_sh.at[dst_v.at[NCHUNK - 1]], add=True)
    plsc.subcore_barrier()
    pltpu.sync_copy(acc_sh.at[pl.ds(sid * NPS, NPS)],
                    out_hbm.at[cid, pl.ds(sid * NPS, NPS)])


def _tc1_body(hist_ref, x_ref, w_ref, dinv_ref, hs_ref):
    deg = jnp.sum(hist_ref[...], axis=1, keepdims=True) + 1.0
    dinv = lax.rsqrt(deg)
    h = jnp.dot(x_ref[...], w_ref[...], preferred_element_type=jnp.float32)
    dinv_ref[...] = dinv
    hs_ref[...] = h * dinv


_tc1 = pl.pallas_call(
    _tc1_body,
    grid=(G,),
    in_specs=[
        pl.BlockSpec((RB, NW), lambda i: (i, 0)),
        pl.BlockSpec((RB, D), lambda i: (i, 0)),
        pl.BlockSpec((D, D), lambda i: (0, 0)),
    ],
    out_specs=[
        pl.BlockSpec((RB, 1), lambda i: (i, 0)),
        pl.BlockSpec((RB, D), lambda i: (i, 0)),
    ],
    out_shape=[
        jax.ShapeDtypeStruct((N_NODES, 1), jnp.float32),
        jax.ShapeDtypeStruct((N_NODES, D), jnp.float32),
    ],
)


def _tc2_body(agg_ref, hs_ref, dinv_ref, b_ref, w_ref, out_ref):
    dinv = dinv_ref[...]
    s = agg_ref[0] + agg_ref[1] + hs_ref[...]
    z = jnp.maximum(dinv * s + b_ref[...], 0.0)
    h2 = jnp.dot(z, w_ref[...], preferred_element_type=jnp.float32)
    out_ref[...] = h2 * dinv


_tc2 = pl.pallas_call(
    _tc2_body,
    grid=(G,),
    in_specs=[
        pl.BlockSpec((NC, RB, D), lambda i: (0, i, 0)),
        pl.BlockSpec((RB, D), lambda i: (i, 0)),
        pl.BlockSpec((RB, 1), lambda i: (i, 0)),
        pl.BlockSpec((1, D), lambda i: (0, 0)),
        pl.BlockSpec((D, D), lambda i: (0, 0)),
    ],
    out_specs=pl.BlockSpec((RB, D), lambda i: (i, 0)),
    out_shape=jax.ShapeDtypeStruct((N_NODES, D), jnp.float32),
)


def _tc3_body(agg_ref, hs_ref, dinv_ref, b_ref, out_ref):
    dinv = dinv_ref[...]
    out_ref[...] = dinv * (agg_ref[0] + agg_ref[1] + hs_ref[...]) + b_ref[...]


_tc3 = pl.pallas_call(
    _tc3_body,
    grid=(G,),
    in_specs=[
        pl.BlockSpec((NC, RB, D), lambda i: (0, i, 0)),
        pl.BlockSpec((RB, D), lambda i: (i, 0)),
        pl.BlockSpec((RB, 1), lambda i: (i, 0)),
        pl.BlockSpec((1, D), lambda i: (0, 0)),
    ],
    out_specs=pl.BlockSpec((RB, D), lambda i: (i, 0)),
    out_shape=jax.ShapeDtypeStruct((N_NODES, D), jnp.float32),
)


def kernel(x, edge_index, W1, b1, W2, b2):
    src = edge_index[0].astype(jnp.int32).reshape(NW * NCHUNK, CH)
    dst = edge_index[1].astype(jnp.int32).reshape(NW * NCHUNK, CH)
    dst_flat = edge_index[1].astype(jnp.int32)

    degree_hist = _make_degree_hist()
    edge_aggregate = _make_edge_aggregate()

    hist = degree_hist(dst_flat)
    dinv, hs1 = _tc1(hist.T, x, W1)
    agg1 = edge_aggregate(hs1, src, dst)
    hs2 = _tc2(agg1, hs1, dinv, b1.reshape(1, D), W2)
    agg2 = edge_aggregate(hs2, src, dst)
    out = _tc3(agg2, hs2, dinv, b2.reshape(1, D))
    return out


# speedup vs baseline: 32.3478x; 1.0347x over previous
import functools

import jax
import jax.numpy as jnp
from jax import lax
from jax.experimental import pallas as pl
from jax.experimental.pallas import tpu as pltpu
from jax.experimental.pallas import tpu_sc as plsc

N_NODES = 10000
D = 128
N_EDGES = 320000

NC = 2
NS = 16
NW = NC * NS
EPW = N_EDGES // NW
CH = 80
NCHUNK = EPW // CH
NPS = N_NODES // NS
RB = 1000
G = N_NODES // RB

@functools.cache
def _make_degree_hist():
    mesh = plsc.VectorSubcoreMesh(core_axis_name="c", subcore_axis_name="s")
    return pl.kernel(
        _degree_hist_body,
        mesh=mesh,
        compiler_params=pltpu.CompilerParams(
            needs_layout_passes=False, use_tc_tiling_on_sc=False),
        out_type=jax.ShapeDtypeStruct((NW, N_NODES), jnp.float32),
        scratch_types=[
            pltpu.VMEM((EPW,), jnp.int32),
            pltpu.VMEM((N_NODES,), jnp.float32),
        ],
    )


def _degree_hist_body(dst_hbm, out_hbm, dst_v, hist_v):
    cid = lax.axis_index("c")
    sid = lax.axis_index("s")
    wid = sid * NC + cid
    pltpu.sync_copy(dst_hbm.at[pl.ds(wid * EPW, EPW)], dst_v)

    zeros = jnp.zeros((16,), jnp.float32)

    def zero_body(i, _):
        hist_v[pl.ds(i * 16, 16)] = zeros
        return ()

    lax.fori_loop(0, N_NODES // 16, zero_body, ())

    ones = jnp.ones((16,), jnp.float32)

    def body(i, _):
        idx = dst_v[pl.ds(i * 16, 16)]
        plsc.addupdate_scatter(hist_v, [idx], ones)
        return ()

    lax.fori_loop(0, EPW // 16, body, ())
    pltpu.sync_copy(hist_v, out_hbm.at[wid])


@functools.cache
def _make_edge_aggregate():
    mesh = plsc.VectorSubcoreMesh(core_axis_name="c", subcore_axis_name="s")
    return pl.kernel(
        _edge_aggregate_body,
        mesh=mesh,
        compiler_params=pltpu.CompilerParams(
            needs_layout_passes=False, use_tc_tiling_on_sc=False),
        out_type=jax.ShapeDtypeStruct((NC, N_NODES, D), jnp.bfloat16),
        scratch_types=[
            pltpu.VMEM((NCHUNK, CH), jnp.int32),
            pltpu.VMEM((NCHUNK, CH), jnp.int32),
            pltpu.VMEM((CH, D), jnp.bfloat16),
            pltpu.VMEM((CH, D), jnp.bfloat16),
            pltpu.VMEM_SHARED((N_NODES, D), jnp.bfloat16),
            pltpu.SemaphoreType.DMA,
            pltpu.SemaphoreType.DMA,
        ],
    )


def _edge_aggregate_body(h_hbm, src_hbm, dst_hbm, out_hbm,
                         src_v, dst_v, rows_a, rows_b, acc_sh,
                         sem_a, sem_b):
    cid = lax.axis_index("c")
    sid = lax.axis_index("s")
    wid = sid * NC + cid

    pltpu.sync_copy(src_hbm.at[pl.ds(wid * NCHUNK, NCHUNK)], src_v)
    pltpu.sync_copy(dst_hbm.at[pl.ds(wid * NCHUNK, NCHUNK)], dst_v)

    zeros = jnp.zeros((32,), jnp.bfloat16)

    def zero_body(i, _):
        rows_a[i // 4, pl.ds((i % 4) * 32, 32)] = zeros
        return ()

    lax.fori_loop(0, CH * (D // 32), zero_body, ())
    for k in range(NPS // CH):
        pltpu.sync_copy(rows_a, acc_sh.at[pl.ds(sid * NPS + k * CH, CH)])
    tail = NPS - (NPS // CH) * CH
    if tail:
        pltpu.sync_copy(rows_a.at[pl.ds(0, tail)],
                        acc_sh.at[pl.ds(sid * NPS + (NPS // CH) * CH, tail)])
    plsc.subcore_barrier()

    def wait_gather(buf, sem):
        pltpu.make_async_copy(h_hbm.at[src_v.at[0]], buf, sem).wait()

    pltpu.async_copy(h_hbm.at[src_v.at[0]], rows_a, sem_a)

    def chunk_pair(i, _):
        j0 = i * 2
        pltpu.async_copy(h_hbm.at[src_v.at[j0 + 1]], rows_b, sem_b)
        wait_gather(rows_a, sem_a)
        pltpu.sync_copy(rows_a, acc_sh.at[dst_v.at[j0]], add=True)
        pltpu.async_copy(h_hbm.at[src_v.at[j0 + 2]], rows_a, sem_a)
        wait_gather(rows_b, sem_b)
        pltpu.sync_copy(rows_b, acc_sh.at[dst_v.at[j0 + 1]], add=True)
        return ()

    lax.fori_loop(0, (NCHUNK - 1) // 2, chunk_pair, ())
    wait_gather(rows_a, sem_a)
    pltpu.sync_copy(rows_a, acc_sh.at[dst_v.at[NCHUNK - 1]], add=True)
    plsc.subcore_barrier()
    pltpu.sync_copy(acc_sh.at[pl.ds(sid * NPS, NPS)],
                    out_hbm.at[cid, pl.ds(sid * NPS, NPS)])


def _tc1_body(hist_ref, x_ref, w_ref, dinv_ref, hs_ref):
    deg = jnp.sum(hist_ref[...], axis=1, keepdims=True) + 1.0
    dinv = lax.rsqrt(deg)
    h = jnp.dot(x_ref[...], w_ref[...], preferred_element_type=jnp.float32)
    dinv_ref[...] = dinv
    hs_ref[...] = (h * dinv).astype(jnp.bfloat16)


_tc1 = pl.pallas_call(
    _tc1_body,
    grid=(G,),
    in_specs=[
        pl.BlockSpec((RB, NW), lambda i: (i, 0)),
        pl.BlockSpec((RB, D), lambda i: (i, 0)),
        pl.BlockSpec((D, D), lambda i: (0, 0)),
    ],
    out_specs=[
        pl.BlockSpec((RB, 1), lambda i: (i, 0)),
        pl.BlockSpec((RB, D), lambda i: (i, 0)),
    ],
    out_shape=[
        jax.ShapeDtypeStruct((N_NODES, 1), jnp.float32),
        jax.ShapeDtypeStruct((N_NODES, D), jnp.bfloat16),
    ],
)


def _tc2_body(agg_ref, hs_ref, dinv_ref, b_ref, w_ref, out_ref):
    dinv = dinv_ref[...]
    s = (agg_ref[0].astype(jnp.float32) + agg_ref[1].astype(jnp.float32)
         + hs_ref[...].astype(jnp.float32))
    z = jnp.maximum(dinv * s + b_ref[...], 0.0)
    h2 = jnp.dot(z, w_ref[...], preferred_element_type=jnp.float32)
    out_ref[...] = (h2 * dinv).astype(jnp.bfloat16)


_tc2 = pl.pallas_call(
    _tc2_body,
    grid=(G,),
    in_specs=[
        pl.BlockSpec((NC, RB, D), lambda i: (0, i, 0)),
        pl.BlockSpec((RB, D), lambda i: (i, 0)),
        pl.BlockSpec((RB, 1), lambda i: (i, 0)),
        pl.BlockSpec((1, D), lambda i: (0, 0)),
        pl.BlockSpec((D, D), lambda i: (0, 0)),
    ],
    out_specs=pl.BlockSpec((RB, D), lambda i: (i, 0)),
    out_shape=jax.ShapeDtypeStruct((N_NODES, D), jnp.bfloat16),
)


def _tc3_body(agg_ref, hs_ref, dinv_ref, b_ref, out_ref):
    dinv = dinv_ref[...]
    s = (agg_ref[0].astype(jnp.float32) + agg_ref[1].astype(jnp.float32)
         + hs_ref[...].astype(jnp.float32))
    out_ref[...] = dinv * s + b_ref[...]


_tc3 = pl.pallas_call(
    _tc3_body,
    grid=(G,),
    in_specs=[
        pl.BlockSpec((NC, RB, D), lambda i: (0, i, 0)),
        pl.BlockSpec((RB, D), lambda i: (i, 0)),
        pl.BlockSpec((RB, 1), lambda i: (i, 0)),
        pl.BlockSpec((1, D), lambda i: (0, 0)),
    ],
    out_specs=pl.BlockSpec((RB, D), lambda i: (i, 0)),
    out_shape=jax.ShapeDtypeStruct((N_NODES, D), jnp.float32),
)


def kernel(x, edge_index, W1, b1, W2, b2):
    src = edge_index[0].astype(jnp.int32).reshape(NW * NCHUNK, CH)
    dst = edge_index[1].astype(jnp.int32).reshape(NW * NCHUNK, CH)
    dst_flat = edge_index[1].astype(jnp.int32)

    degree_hist = _make_degree_hist()
    edge_aggregate = _make_edge_aggregate()

    hist = degree_hist(dst_flat)
    dinv, hs1 = _tc1(hist.T, x, W1)
    agg1 = edge_aggregate(hs1, src, dst)
    hs2 = _tc2(agg1, hs1, dinv, b1.reshape(1, D), W2)
    agg2 = edge_aggregate(hs2, src, dst)
    out = _tc3(agg2, hs2, dinv, b2.reshape(1, D))
    return out


# speedup vs baseline: 33.1295x; 1.0242x over previous
import functools

import jax
import jax.numpy as jnp
from jax import lax
from jax.experimental import pallas as pl
from jax.experimental.pallas import tpu as pltpu
from jax.experimental.pallas import tpu_sc as plsc

N_NODES = 10000
D = 128
N_EDGES = 320000

NC = 2
NS = 16
NW = NC * NS
EPW = N_EDGES // NW
CH = 80
NCHUNK = EPW // CH
NPS = N_NODES // NS
RB = 2000
G = N_NODES // RB

@functools.cache
def _make_degree_hist():
    mesh = plsc.VectorSubcoreMesh(core_axis_name="c", subcore_axis_name="s")
    return pl.kernel(
        _degree_hist_body,
        mesh=mesh,
        compiler_params=pltpu.CompilerParams(
            needs_layout_passes=False, use_tc_tiling_on_sc=False),
        out_type=jax.ShapeDtypeStruct((NW, N_NODES), jnp.float32),
        scratch_types=[
            pltpu.VMEM((EPW,), jnp.int32),
            pltpu.VMEM((N_NODES,), jnp.float32),
        ],
    )


def _degree_hist_body(dst_hbm, out_hbm, dst_v, hist_v):
    cid = lax.axis_index("c")
    sid = lax.axis_index("s")
    wid = sid * NC + cid
    pltpu.sync_copy(dst_hbm.at[pl.ds(wid * EPW, EPW)], dst_v)

    zeros = jnp.zeros((16,), jnp.float32)

    def zero_body(i, _):
        hist_v[pl.ds(i * 16, 16)] = zeros
        return ()

    lax.fori_loop(0, N_NODES // 16, zero_body, ())

    ones = jnp.ones((16,), jnp.float32)

    def body(i, _):
        idx = dst_v[pl.ds(i * 16, 16)]
        plsc.addupdate_scatter(hist_v, [idx], ones)
        return ()

    lax.fori_loop(0, EPW // 16, body, ())
    pltpu.sync_copy(hist_v, out_hbm.at[wid])


@functools.cache
def _make_edge_aggregate():
    mesh = plsc.VectorSubcoreMesh(core_axis_name="c", subcore_axis_name="s")
    return pl.kernel(
        _edge_aggregate_body,
        mesh=mesh,
        compiler_params=pltpu.CompilerParams(
            needs_layout_passes=False, use_tc_tiling_on_sc=False),
        out_type=jax.ShapeDtypeStruct((NC, N_NODES, D), jnp.bfloat16),
        scratch_types=[
            pltpu.VMEM((NCHUNK, CH), jnp.int32),
            pltpu.VMEM((NCHUNK, CH), jnp.int32),
            pltpu.VMEM((CH, D), jnp.bfloat16),
            pltpu.VMEM((CH, D), jnp.bfloat16),
            pltpu.VMEM_SHARED((N_NODES, D), jnp.bfloat16),
            pltpu.SemaphoreType.DMA,
            pltpu.SemaphoreType.DMA,
        ],
    )


def _edge_aggregate_body(h_hbm, src_hbm, dst_hbm, out_hbm,
                         src_v, dst_v, rows_a, rows_b, acc_sh,
                         sem_a, sem_b):
    cid = lax.axis_index("c")
    sid = lax.axis_index("s")
    wid = sid * NC + cid

    pltpu.sync_copy(src_hbm.at[pl.ds(wid * NCHUNK, NCHUNK)], src_v)
    pltpu.sync_copy(dst_hbm.at[pl.ds(wid * NCHUNK, NCHUNK)], dst_v)

    zeros = jnp.zeros((32,), jnp.bfloat16)

    def zero_body(i, _):
        rows_a[i // 4, pl.ds((i % 4) * 32, 32)] = zeros
        return ()

    lax.fori_loop(0, CH * (D // 32), zero_body, ())
    for k in range(NPS // CH):
        pltpu.sync_copy(rows_a, acc_sh.at[pl.ds(sid * NPS + k * CH, CH)])
    tail = NPS - (NPS // CH) * CH
    if tail:
        pltpu.sync_copy(rows_a.at[pl.ds(0, tail)],
                        acc_sh.at[pl.ds(sid * NPS + (NPS // CH) * CH, tail)])
    plsc.subcore_barrier()

    def wait_gather(buf, sem):
        pltpu.make_async_copy(h_hbm.at[src_v.at[0]], buf, sem).wait()

    pltpu.async_copy(h_hbm.at[src_v.at[0]], rows_a, sem_a)

    def chunk_pair(i, _):
        j0 = i * 2
        pltpu.async_copy(h_hbm.at[src_v.at[j0 + 1]], rows_b, sem_b)
        wait_gather(rows_a, sem_a)
        pltpu.sync_copy(rows_a, acc_sh.at[dst_v.at[j0]], add=True)
        pltpu.async_copy(h_hbm.at[src_v.at[j0 + 2]], rows_a, sem_a)
        wait_gather(rows_b, sem_b)
        pltpu.sync_copy(rows_b, acc_sh.at[dst_v.at[j0 + 1]], add=True)
        return ()

    lax.fori_loop(0, (NCHUNK - 1) // 2, chunk_pair, ())
    wait_gather(rows_a, sem_a)
    pltpu.sync_copy(rows_a, acc_sh.at[dst_v.at[NCHUNK - 1]], add=True)
    plsc.subcore_barrier()
    pltpu.sync_copy(acc_sh.at[pl.ds(sid * NPS, NPS)],
                    out_hbm.at[cid, pl.ds(sid * NPS, NPS)])


def _tc1_body(hist_ref, x_ref, w_ref, dinv_ref, hs_ref):
    deg = jnp.sum(hist_ref[...], axis=1, keepdims=True) + 1.0
    dinv = lax.rsqrt(deg)
    h = jnp.dot(x_ref[...], w_ref[...], preferred_element_type=jnp.float32)
    dinv_ref[...] = dinv
    hs_ref[...] = (h * dinv).astype(jnp.bfloat16)


_tc1 = pl.pallas_call(
    _tc1_body,
    grid=(G,),
    in_specs=[
        pl.BlockSpec((RB, NW), lambda i: (i, 0)),
        pl.BlockSpec((RB, D), lambda i: (i, 0)),
        pl.BlockSpec((D, D), lambda i: (0, 0)),
    ],
    out_specs=[
        pl.BlockSpec((RB, 1), lambda i: (i, 0)),
        pl.BlockSpec((RB, D), lambda i: (i, 0)),
    ],
    out_shape=[
        jax.ShapeDtypeStruct((N_NODES, 1), jnp.float32),
        jax.ShapeDtypeStruct((N_NODES, D), jnp.bfloat16),
    ],
)


def _tc2_body(agg_ref, hs_ref, dinv_ref, b_ref, w_ref, out_ref):
    dinv = dinv_ref[...]
    s = (agg_ref[0].astype(jnp.float32) + agg_ref[1].astype(jnp.float32)
         + hs_ref[...].astype(jnp.float32))
    z = jnp.maximum(dinv * s + b_ref[...], 0.0)
    h2 = jnp.dot(z, w_ref[...], preferred_element_type=jnp.float32)
    out_ref[...] = (h2 * dinv).astype(jnp.bfloat16)


_tc2 = pl.pallas_call(
    _tc2_body,
    grid=(G,),
    in_specs=[
        pl.BlockSpec((NC, RB, D), lambda i: (0, i, 0)),
        pl.BlockSpec((RB, D), lambda i: (i, 0)),
        pl.BlockSpec((RB, 1), lambda i: (i, 0)),
        pl.BlockSpec((1, D), lambda i: (0, 0)),
        pl.BlockSpec((D, D), lambda i: (0, 0)),
    ],
    out_specs=pl.BlockSpec((RB, D), lambda i: (i, 0)),
    out_shape=jax.ShapeDtypeStruct((N_NODES, D), jnp.bfloat16),
)


def _tc3_body(agg_ref, hs_ref, dinv_ref, b_ref, out_ref):
    dinv = dinv_ref[...]
    s = (agg_ref[0].astype(jnp.float32) + agg_ref[1].astype(jnp.float32)
         + hs_ref[...].astype(jnp.float32))
    out_ref[...] = dinv * s + b_ref[...]


_tc3 = pl.pallas_call(
    _tc3_body,
    grid=(G,),
    in_specs=[
        pl.BlockSpec((NC, RB, D), lambda i: (0, i, 0)),
        pl.BlockSpec((RB, D), lambda i: (i, 0)),
        pl.BlockSpec((RB, 1), lambda i: (i, 0)),
        pl.BlockSpec((1, D), lambda i: (0, 0)),
    ],
    out_specs=pl.BlockSpec((RB, D), lambda i: (i, 0)),
    out_shape=jax.ShapeDtypeStruct((N_NODES, D), jnp.float32),
)


def kernel(x, edge_index, W1, b1, W2, b2):
    src = edge_index[0].astype(jnp.int32).reshape(NW * NCHUNK, CH)
    dst = edge_index[1].astype(jnp.int32).reshape(NW * NCHUNK, CH)
    dst_flat = edge_index[1].astype(jnp.int32)

    degree_hist = _make_degree_hist()
    edge_aggregate = _make_edge_aggregate()

    hist = degree_hist(dst_flat)
    dinv, hs1 = _tc1(hist.T, x, W1)
    agg1 = edge_aggregate(hs1, src, dst)
    hs2 = _tc2(agg1, hs1, dinv, b1.reshape(1, D), W2)
    agg2 = edge_aggregate(hs2, src, dst)
    out = _tc3(agg2, hs2, dinv, b2.reshape(1, D))
    return out
